# baseline (speedup 1.0000x reference)
import math

import jax
import jax.numpy as jnp
from jax.experimental import pallas as pl
from jax.experimental.pallas import tpu as pltpu

_B, _N, _C = 2, 4096, 4096
_E = 64
_K = 8
_CF = 1.25
_T = _B * _N
_BT = 512
_CAP = math.ceil(_CF * _T * _K / _E)
_KE = _K * _E


def _router_kernel(x_ref, wt_ref, b_ref, l_ref, disp_ref, comb_ref, cnt_ref):
    i = pl.program_id(0)

    @pl.when(i == 0)
    def _init():
        cnt_ref[...] = jnp.zeros_like(cnt_ref)

    x = x_ref[...]
    logits = jnp.dot(x, wt_ref[...],
                     preferred_element_type=jnp.float32) + b_ref[...]
    m = jnp.max(logits, axis=1, keepdims=True)
    ex = jnp.exp(logits - m)

    iota = jax.lax.broadcasted_iota(jnp.int32, (_BT, _E), 1)
    bits = jax.lax.bitcast_convert_type(ex, jnp.int32)
    v = jnp.bitwise_or(jnp.bitwise_and(bits, -64), (_E - 1) - iota)

    for r in range(_K):
        mxi = jnp.max(v, axis=1, keepdims=True)
        ohb = v == mxi
        v = jnp.where(ohb, -(2 + r), v)
    oh8, mx8 = ohb, mxi

    mx9 = jnp.max(v, axis=1, keepdims=True)
    oh9 = v == mx9
    exa = jnp.sum(jnp.where(oh8, ex, 0.0), axis=1, keepdims=True)
    exb = jnp.sum(jnp.where(oh9, ex, 0.0), axis=1, keepdims=True)
    idxa = jnp.bitwise_and(mx8, _E - 1)
    idxb = jnp.bitwise_and(mx9, _E - 1)
    swap = (exb > exa) | ((exb == exa) & (idxb > idxa))
    v = jnp.where(swap & oh8, 0, v)
    v = jnp.where(swap & oh9, -(2 + _K - 1), v)

    support = v <= -2

    ohp = jnp.concatenate(
        [jnp.where(v == -(2 + r), 1.0, 0.0) for r in range(_K)],
        axis=1).astype(jnp.bfloat16)
    cum = jnp.dot(l_ref[...], ohp,
                  preferred_element_type=jnp.float32)
    cnt_prev = cnt_ref[...]
    cnt_ref[...] = cnt_prev + cum[_BT - 1 : _BT, :]

    accb = (cum <= (_CAP - cnt_prev)) & (ohp > 0)
    acc = jnp.where(accb, 1.0, 0.0)
    a4 = acc[:, : 4 * _E] + acc[:, 4 * _E :]
    a2 = a4[:, : 2 * _E] + a4[:, 2 * _E :]
    disp = a2[:, :_E] + a2[:, _E:]

    comb_raw = jnp.where(disp > 0, ex, 0.0)
    s8 = jnp.sum(jnp.where(support, ex, 0.0), axis=1, keepdims=True)
    sa = jnp.sum(comb_raw, axis=1, keepdims=True)
    d1 = jnp.maximum(s8, 1e-6)
    f = 1.0 / (d1 * jnp.maximum(sa / d1, 1e-6))
    disp_ref[...] = disp
    comb_ref[...] = comb_raw * f


def kernel(x, W, b):
    xf = x.reshape(_T, _C)
    wt = W.T
    b2 = b.reshape(1, _E)
    row = jax.lax.broadcasted_iota(jnp.int32, (_BT, _BT), 0)
    col = jax.lax.broadcasted_iota(jnp.int32, (_BT, _BT), 1)
    ltri = (row >= col).astype(jnp.bfloat16)
    disp, comb = pl.pallas_call(
        _router_kernel,
        grid=(_T // _BT,),
        in_specs=[
            pl.BlockSpec((_BT, _C), lambda i: (i, 0)),
            pl.BlockSpec((_C, _E), lambda i: (0, 0)),
            pl.BlockSpec((1, _E), lambda i: (0, 0)),
            pl.BlockSpec((_BT, _BT), lambda i: (0, 0)),
        ],
        out_specs=[
            pl.BlockSpec((_BT, _E), lambda i: (i, 0)),
            pl.BlockSpec((_BT, _E), lambda i: (i, 0)),
        ],
        out_shape=[
            jax.ShapeDtypeStruct((_T, _E), jnp.float32),
            jax.ShapeDtypeStruct((_T, _E), jnp.float32),
        ],
        scratch_shapes=[pltpu.VMEM((1, _KE), jnp.float32)],
        compiler_params=pltpu.CompilerParams(
            dimension_semantics=("arbitrary",),
        ),
    )(xf, wt, b2, ltri)
    return disp.reshape(_B, _N, _E), comb.reshape(_B, _N, _E)


# speedup vs baseline: 1.0904x; 1.0904x over previous
import math

import jax
import jax.numpy as jnp
from jax.experimental import pallas as pl
from jax.experimental.pallas import tpu as pltpu

_B, _N, _C = 2, 4096, 4096
_E = 64
_K = 8
_CF = 1.25
_T = _B * _N
_BT = 512
_CAP = math.ceil(_CF * _T * _K / _E)
_KE = _K * _E


def _router_kernel(x_ref, wt_ref, b_ref, disp_ref, comb_ref, cnt_ref):
    i = pl.program_id(0)

    @pl.when(i == 0)
    def _init():
        cnt_ref[...] = jnp.zeros_like(cnt_ref)

    x = x_ref[...]
    logits = jnp.dot(x, wt_ref[...],
                     preferred_element_type=jnp.float32) + b_ref[...]
    m = jnp.max(logits, axis=1, keepdims=True)
    ex = jnp.exp(logits - m)

    iota = jax.lax.broadcasted_iota(jnp.int32, (_BT, _E), 1)
    bits = jax.lax.bitcast_convert_type(ex, jnp.int32)
    v = jnp.bitwise_or(jnp.bitwise_and(bits, -64), (_E - 1) - iota)

    for r in range(_K):
        mxi = jnp.max(v, axis=1, keepdims=True)
        ohb = v == mxi
        v = jnp.where(ohb, -(2 + r), v)
    oh8, mx8 = ohb, mxi

    mx9 = jnp.max(v, axis=1, keepdims=True)
    oh9 = v == mx9
    exa = jnp.sum(jnp.where(oh8, ex, 0.0), axis=1, keepdims=True)
    exb = jnp.sum(jnp.where(oh9, ex, 0.0), axis=1, keepdims=True)
    idxa = jnp.bitwise_and(mx8, _E - 1)
    idxb = jnp.bitwise_and(mx9, _E - 1)
    swap = (exb > exa) | ((exb == exa) & (idxb > idxa))
    v = jnp.where(swap & oh8, 0, v)
    v = jnp.where(swap & oh9, -(2 + _K - 1), v)

    support = v <= -2

    ohp = jnp.concatenate(
        [jnp.where(v == -(2 + r), 1.0, 0.0) for r in range(_K)], axis=1)
    cum = ohp
    sft = 1
    while sft < _BT:
        cum = cum + jnp.concatenate(
            [jnp.zeros((sft, _KE), jnp.float32), cum[: _BT - sft, :]], axis=0)
        sft *= 2
    cnt_prev = cnt_ref[...]
    cnt_ref[...] = cnt_prev + cum[_BT - 1 : _BT, :]

    acc = jnp.where((cum <= (_CAP - cnt_prev)) & (ohp > 0), 1.0, 0.0)
    a4 = acc[:, : 4 * _E] + acc[:, 4 * _E :]
    a2 = a4[:, : 2 * _E] + a4[:, 2 * _E :]
    disp = a2[:, :_E] + a2[:, _E:]

    comb_raw = jnp.where(disp > 0, ex, 0.0)
    s8 = jnp.sum(jnp.where(support, ex, 0.0), axis=1, keepdims=True)
    sa = jnp.sum(comb_raw, axis=1, keepdims=True)
    d1 = jnp.maximum(s8, 1e-6)
    f = 1.0 / (d1 * jnp.maximum(sa / d1, 1e-6))
    disp_ref[...] = disp
    comb_ref[...] = comb_raw * f


def kernel(x, W, b):
    xf = x.reshape(_T, _C)
    wt = W.T
    b2 = b.reshape(1, _E)
    disp, comb = pl.pallas_call(
        _router_kernel,
        grid=(_T // _BT,),
        in_specs=[
            pl.BlockSpec((_BT, _C), lambda i: (i, 0)),
            pl.BlockSpec((_C, _E), lambda i: (0, 0)),
            pl.BlockSpec((1, _E), lambda i: (0, 0)),
        ],
        out_specs=[
            pl.BlockSpec((_BT, _E), lambda i: (i, 0)),
            pl.BlockSpec((_BT, _E), lambda i: (i, 0)),
        ],
        out_shape=[
            jax.ShapeDtypeStruct((_T, _E), jnp.float32),
            jax.ShapeDtypeStruct((_T, _E), jnp.float32),
        ],
        scratch_shapes=[pltpu.VMEM((1, _KE), jnp.float32)],
        compiler_params=pltpu.CompilerParams(
            dimension_semantics=("arbitrary",),
        ),
    )(xf, wt, b2)
    return disp.reshape(_B, _N, _E), comb.reshape(_B, _N, _E)


# speedup vs baseline: 1.3056x; 1.1973x over previous
import math

import jax
import jax.numpy as jnp
from jax.experimental import pallas as pl
from jax.experimental.pallas import tpu as pltpu

_B, _N, _C = 2, 4096, 4096
_E = 64
_K = 8
_CF = 1.25
_T = _B * _N
_BT = 512
_BH = _BT // 2
_CAP = math.ceil(_CF * _T * _K / _E)
_KE = _K * _E


def _router_kernel(xa_ref, xb_ref, wt_ref, b_ref, disp_ref, comb_ref,
                   cnt_ref):
    i = pl.program_id(0)

    @pl.when(i == 0)
    def _init():
        cnt_ref[...] = jnp.zeros_like(cnt_ref)

    wt = wt_ref[...]
    logits = jnp.concatenate(
        [jnp.dot(xa_ref[...], wt, preferred_element_type=jnp.float32),
         jnp.dot(xb_ref[...], wt, preferred_element_type=jnp.float32)],
        axis=0) + b_ref[...]
    m = jnp.max(logits, axis=1, keepdims=True)
    ex = jnp.exp(logits - m)

    iota = jax.lax.broadcasted_iota(jnp.int32, (_BT, _E), 1)
    bits = jax.lax.bitcast_convert_type(ex, jnp.int32)
    v = jnp.bitwise_or(jnp.bitwise_and(bits, -64), (_E - 1) - iota)

    for r in range(_K):
        mxi = jnp.max(v, axis=1, keepdims=True)
        ohb = v == mxi
        v = jnp.where(ohb, -(2 + r), v)
    oh8, mx8 = ohb, mxi

    mx9 = jnp.max(v, axis=1, keepdims=True)
    oh9 = v == mx9
    exa = jnp.sum(jnp.where(oh8, ex, 0.0), axis=1, keepdims=True)
    exb = jnp.sum(jnp.where(oh9, ex, 0.0), axis=1, keepdims=True)
    idxa = jnp.bitwise_and(mx8, _E - 1)
    idxb = jnp.bitwise_and(mx9, _E - 1)
    swap = (exb > exa) | ((exb == exa) & (idxb > idxa))
    v = jnp.where(swap & oh8, 0, v)
    v = jnp.where(swap & oh9, -(2 + _K - 1), v)

    support = v <= -2

    colsums = jnp.concatenate(
        [jnp.sum(jnp.where(v == -(2 + r), 1.0, 0.0), axis=0, keepdims=True)
         for r in range(_K)], axis=1)
    cnt_prev = cnt_ref[...]
    cnt_ref[...] = cnt_prev + colsums

    disp = jnp.where(support, 1.0, 0.0)
    comb_raw = jnp.where(support, ex, 0.0)
    s8 = jnp.sum(comb_raw, axis=1, keepdims=True)
    d1 = jnp.maximum(s8, 1e-6)
    f = 1.0 / (d1 * jnp.maximum(s8 / d1, 1e-6))
    disp_ref[...] = disp
    comb_ref[...] = comb_raw * f

    @pl.when(jnp.max(cnt_prev + colsums) > _CAP)
    def _slow():
        ohp = jnp.concatenate(
            [jnp.where(v == -(2 + r), 1.0, 0.0) for r in range(_K)], axis=1)
        cum = ohp
        sft = 1
        while sft < _BT:
            cum = cum + jnp.concatenate(
                [jnp.zeros((sft, _KE), jnp.float32), cum[: _BT - sft, :]],
                axis=0)
            sft *= 2
        acc = jnp.where((cum <= (_CAP - cnt_prev)) & (ohp > 0), 1.0, 0.0)
        a4 = acc[:, : 4 * _E] + acc[:, 4 * _E :]
        a2 = a4[:, : 2 * _E] + a4[:, 2 * _E :]
        disp2 = a2[:, :_E] + a2[:, _E:]
        comb_raw2 = jnp.where(disp2 > 0, ex, 0.0)
        s8b = jnp.sum(jnp.where(support, ex, 0.0), axis=1, keepdims=True)
        sa = jnp.sum(comb_raw2, axis=1, keepdims=True)
        d1b = jnp.maximum(s8b, 1e-6)
        fb = 1.0 / (d1b * jnp.maximum(sa / d1b, 1e-6))
        disp_ref[...] = disp2
        comb_ref[...] = comb_raw2 * fb


def kernel(x, W, b):
    xf = x.reshape(_T, _C)
    wt = W.T
    b2 = b.reshape(1, _E)
    disp, comb = pl.pallas_call(
        _router_kernel,
        grid=(_T // _BT,),
        in_specs=[
            pl.BlockSpec((_BH, _C), lambda i: (2 * i, 0)),
            pl.BlockSpec((_BH, _C), lambda i: (2 * i + 1, 0)),
            pl.BlockSpec((_C, _E), lambda i: (0, 0)),
            pl.BlockSpec((1, _E), lambda i: (0, 0)),
        ],
        out_specs=[
            pl.BlockSpec((_BT, _E), lambda i: (i, 0)),
            pl.BlockSpec((_BT, _E), lambda i: (i, 0)),
        ],
        out_shape=[
            jax.ShapeDtypeStruct((_T, _E), jnp.float32),
            jax.ShapeDtypeStruct((_T, _E), jnp.float32),
        ],
        scratch_shapes=[pltpu.VMEM((1, _KE), jnp.float32)],
        compiler_params=pltpu.CompilerParams(
            dimension_semantics=("arbitrary",),
        ),
    )(xf, xf, wt, b2)
    return disp.reshape(_B, _N, _E), comb.reshape(_B, _N, _E)


# speedup vs baseline: 1.4746x; 1.1294x over previous
import math

import jax
import jax.numpy as jnp
from jax.experimental import pallas as pl
from jax.experimental.pallas import tpu as pltpu

_B, _N, _C = 2, 4096, 4096
_E = 64
_K = 8
_CF = 1.25
_T = _B * _N
_BT = 1024
_BH = _BT // 2
_CAP = math.ceil(_CF * _T * _K / _E)
_KE = _K * _E


def _router_kernel(xa_ref, xb_ref, wt_ref, b_ref, disp_ref, comb_ref,
                   cnt_ref):
    i = pl.program_id(0)

    @pl.when(i == 0)
    def _init():
        cnt_ref[...] = jnp.zeros_like(cnt_ref)

    wt = wt_ref[...]
    logits = jnp.concatenate(
        [jnp.dot(xa_ref[...], wt, preferred_element_type=jnp.float32),
         jnp.dot(xb_ref[...], wt, preferred_element_type=jnp.float32)],
        axis=0) + b_ref[...]
    m = jnp.max(logits, axis=1, keepdims=True)
    ex = jnp.exp(logits - m)

    iota = jax.lax.broadcasted_iota(jnp.int32, (_BT, _E), 1)
    bits = jax.lax.bitcast_convert_type(ex, jnp.int32)
    v = jnp.bitwise_or(jnp.bitwise_and(bits, -64), (_E - 1) - iota)

    for r in range(_K):
        mxi = jnp.max(v, axis=1, keepdims=True)
        ohb = v == mxi
        v = jnp.where(ohb, -(2 + r), v)
    oh8, mx8 = ohb, mxi

    mx9 = jnp.max(v, axis=1, keepdims=True)
    oh9 = v == mx9
    exa = jnp.sum(jnp.where(oh8, ex, 0.0), axis=1, keepdims=True)
    exb = jnp.sum(jnp.where(oh9, ex, 0.0), axis=1, keepdims=True)
    idxa = jnp.bitwise_and(mx8, _E - 1)
    idxb = jnp.bitwise_and(mx9, _E - 1)
    swap = (exb > exa) | ((exb == exa) & (idxb > idxa))
    v = jnp.where(swap & oh8, 0, v)
    v = jnp.where(swap & oh9, -(2 + _K - 1), v)

    support = v <= -2

    colsums = jnp.concatenate(
        [jnp.sum(jnp.where(v == -(2 + r), 1.0, 0.0), axis=0, keepdims=True)
         for r in range(_K)], axis=1)
    cnt_prev = cnt_ref[...]
    cnt_ref[...] = cnt_prev + colsums

    disp = jnp.where(support, 1.0, 0.0)
    comb_raw = jnp.where(support, ex, 0.0)
    s8 = jnp.sum(comb_raw, axis=1, keepdims=True)
    d1 = jnp.maximum(s8, 1e-6)
    f = 1.0 / (d1 * jnp.maximum(s8 / d1, 1e-6))
    disp_ref[...] = disp
    comb_ref[...] = comb_raw * f

    @pl.when(jnp.max(cnt_prev + colsums) > _CAP)
    def _slow():
        ohp = jnp.concatenate(
            [jnp.where(v == -(2 + r), 1.0, 0.0) for r in range(_K)], axis=1)
        cum = ohp
        sft = 1
        while sft < _BT:
            cum = cum + jnp.concatenate(
                [jnp.zeros((sft, _KE), jnp.float32), cum[: _BT - sft, :]],
                axis=0)
            sft *= 2
        acc = jnp.where((cum <= (_CAP - cnt_prev)) & (ohp > 0), 1.0, 0.0)
        a4 = acc[:, : 4 * _E] + acc[:, 4 * _E :]
        a2 = a4[:, : 2 * _E] + a4[:, 2 * _E :]
        disp2 = a2[:, :_E] + a2[:, _E:]
        comb_raw2 = jnp.where(disp2 > 0, ex, 0.0)
        s8b = jnp.sum(jnp.where(support, ex, 0.0), axis=1, keepdims=True)
        sa = jnp.sum(comb_raw2, axis=1, keepdims=True)
        d1b = jnp.maximum(s8b, 1e-6)
        fb = 1.0 / (d1b * jnp.maximum(sa / d1b, 1e-6))
        disp_ref[...] = disp2
        comb_ref[...] = comb_raw2 * fb


def kernel(x, W, b):
    xf = x.reshape(_T, _C)
    wt = W.T
    b2 = b.reshape(1, _E)
    disp, comb = pl.pallas_call(
        _router_kernel,
        grid=(_T // _BT,),
        in_specs=[
            pl.BlockSpec((_BH, _C), lambda i: (2 * i, 0)),
            pl.BlockSpec((_BH, _C), lambda i: (2 * i + 1, 0)),
            pl.BlockSpec((_C, _E), lambda i: (0, 0)),
            pl.BlockSpec((1, _E), lambda i: (0, 0)),
        ],
        out_specs=[
            pl.BlockSpec((_BT, _E), lambda i: (i, 0)),
            pl.BlockSpec((_BT, _E), lambda i: (i, 0)),
        ],
        out_shape=[
            jax.ShapeDtypeStruct((_T, _E), jnp.float32),
            jax.ShapeDtypeStruct((_T, _E), jnp.float32),
        ],
        scratch_shapes=[pltpu.VMEM((1, _KE), jnp.float32)],
        compiler_params=pltpu.CompilerParams(
            dimension_semantics=("arbitrary",),
        ),
    )(xf, xf, wt, b2)
    return disp.reshape(_B, _N, _E), comb.reshape(_B, _N, _E)
